# baseline (speedup 1.0000x reference)
import functools

import jax
import jax.numpy as jnp
from jax import lax
from jax.experimental import pallas as pl
from jax.experimental.pallas import tpu as pltpu
from jax.experimental.pallas import tpu_sc as plsc

_T, _D, _H, _E = 16384, 768, 768, 8
_KTOP = 2
_C = (_KTOP * _T) // _E
_TRASH = _E * _C
_BG = 512
_NB = _T // _BG
_BM = 512


def _gate_body(ksh_ref, x_ref, wg_ref,
               idx0_ref, sidx0_ref, w0_ref, e1_ref, loc1p_ref, g1_ref,
               tot0_ref, laux_ref,
               carry0, carry1, me_acc):
    b = pl.program_id(0)

    @pl.when(b == 0)
    def _init():
        carry0[...] = jnp.zeros_like(carry0)
        carry1[...] = jnp.zeros_like(carry1)
        me_acc[...] = jnp.zeros_like(me_acc)

    xb = x_ref[...] + ksh_ref[0]
    logits = jnp.dot(xb, wg_ref[...], preferred_element_type=jnp.float32)
    m = jnp.max(logits, axis=-1, keepdims=True)
    p = jnp.exp(logits - m)
    sc = p / jnp.sum(p, axis=-1, keepdims=True)
    me_acc[...] += jnp.sum(sc, axis=0, keepdims=True)

    lane = lax.broadcasted_iota(jnp.int32, (_BG, _E), 1)
    v0 = jnp.max(sc, axis=-1, keepdims=True)
    i0 = jnp.min(jnp.where(sc == v0, lane, _E), axis=-1, keepdims=True)
    oh0 = (lane == i0).astype(jnp.float32)
    scm = jnp.where(lane == i0, -jnp.inf, sc)
    v1 = jnp.max(scm, axis=-1, keepdims=True)
    i1 = jnp.min(jnp.where(scm == v1, lane, _E), axis=-1, keepdims=True)
    oh1 = (lane == i1).astype(jnp.float32)

    denom = v0 + v1 + 1e-9
    g0 = v0 / denom
    g1 = v1 / denom

    r = lax.broadcasted_iota(jnp.int32, (_BG, _BG), 0)
    c = lax.broadcasted_iota(jnp.int32, (_BG, _BG), 1)
    tril = (r >= c).astype(jnp.float32)
    cum0 = jnp.dot(tril, oh0, preferred_element_type=jnp.float32)
    cum1 = jnp.dot(tril, oh1, preferred_element_type=jnp.float32)
    loc0 = jnp.sum((cum0 - 1.0 + carry0[...]) * oh0, axis=-1, keepdims=True)
    loc1 = jnp.sum((cum1 - 1.0 + carry1[...]) * oh1, axis=-1, keepdims=True)
    carry0[...] += jnp.sum(oh0, axis=0, keepdims=True)
    carry1[...] += jnp.sum(oh1, axis=0, keepdims=True)

    keep0 = (loc0 < _C).astype(jnp.float32)
    idx0 = i0 * _C + jnp.clip(loc0.astype(jnp.int32), 0, _C - 1)
    idx0_ref[...] = idx0
    sidx0_ref[...] = jnp.where(loc0 < _C, idx0, _TRASH)
    w0_ref[...] = jnp.broadcast_to(g0 * keep0, (_BG, 16))
    e1_ref[...] = i1
    loc1p_ref[...] = loc1.astype(jnp.int32)
    g1_ref[...] = g1

    @pl.when(b == _NB - 1)
    def _fin():
        tot0_ref[...] = carry0[...].astype(jnp.int32)
        laux_ref[...] = jnp.sum((me_acc[...] / _T) * (carry0[...] / _T),
                                axis=-1, keepdims=True) * (_E * _E)


def _gate_stage(x, wg, ksh):
    col_i = jax.ShapeDtypeStruct((_T, 1), jnp.int32)
    col_f = jax.ShapeDtypeStruct((_T, 1), jnp.float32)
    bc_f = jax.ShapeDtypeStruct((_T, 16), jnp.float32)
    colspec = pl.BlockSpec((_BG, 1), lambda b: (b, 0))
    bcspec = pl.BlockSpec((_BG, 16), lambda b: (b, 0))
    fullspec8 = pl.BlockSpec((1, _E), lambda b: (0, 0))
    return pl.pallas_call(
        _gate_body,
        grid=(_NB,),
        in_specs=[
            pl.BlockSpec(memory_space=pltpu.SMEM),
            pl.BlockSpec((_BG, _D), lambda b: (b, 0)),
            pl.BlockSpec((_D, _E), lambda b: (0, 0)),
        ],
        out_specs=[colspec, colspec, bcspec, colspec, colspec, colspec,
                   fullspec8, pl.BlockSpec((1, 1), lambda b: (0, 0))],
        out_shape=[col_i, col_i, bc_f, col_i, col_i, col_f,
                   jax.ShapeDtypeStruct((1, _E), jnp.int32),
                   jax.ShapeDtypeStruct((1, 1), jnp.float32)],
        scratch_shapes=[pltpu.VMEM((1, _E), jnp.float32),
                        pltpu.VMEM((1, _E), jnp.float32),
                        pltpu.VMEM((1, _E), jnp.float32)],
    )(ksh, x, wg)


def _ffn_body(ksh_ref, disp_ref, w1_ref, w2_ref, b1_ref, b2_ref, y_ref,
              bias_s):
    m = pl.program_id(1)
    w1 = w1_ref[0]

    @pl.when(m == 0)
    def _():
        bias_s[...] = b1_ref[0] + ksh_ref[0] * jnp.sum(w1, axis=0, keepdims=True)

    a = disp_ref[...]
    h = jnp.dot(a.astype(jnp.bfloat16), w1.astype(jnp.bfloat16),
                preferred_element_type=jnp.float32)
    h = jnp.maximum(h + bias_s[...], 0.0)
    y = jnp.dot(h.astype(jnp.bfloat16), w2_ref[0].astype(jnp.bfloat16),
                preferred_element_type=jnp.float32)
    y_ref[...] = y + b2_ref[0]


def _ffn(disp, fc1_w, fc2_w, fc1_b, fc2_b, ksh):
    nm = _C // _BM
    return pl.pallas_call(
        _ffn_body,
        grid=(_E, nm),
        in_specs=[
            pl.BlockSpec(memory_space=pltpu.SMEM),
            pl.BlockSpec((_BM, _D), lambda e, m: (e * nm + m, 0)),
            pl.BlockSpec((1, _D, _H), lambda e, m: (e, 0, 0)),
            pl.BlockSpec((1, _H, _D), lambda e, m: (e, 0, 0)),
            pl.BlockSpec((1, 1, _H), lambda e, m: (e, 0, 0)),
            pl.BlockSpec((1, 1, _D), lambda e, m: (e, 0, 0)),
        ],
        out_specs=pl.BlockSpec((_BM, _D), lambda e, m: (e * nm + m, 0)),
        out_shape=jax.ShapeDtypeStruct((_E * _C, _D), jnp.float32),
        scratch_shapes=[pltpu.VMEM((1, _H), jnp.float32)],
    )(ksh, disp, fc1_w, fc2_w, fc1_b, fc2_b)


_B2 = 2048
_NB2 = _T // _B2


def _route1_body(e1_ref, l1_ref, g1_ref, tot0_ref,
                 idx1_ref, sidx1_ref, w1_ref):
    e1 = e1_ref[...]
    lane = lax.broadcasted_iota(jnp.int32, (_B2, _E), 1)
    t0 = jnp.sum(jnp.where(e1 == lane, tot0_ref[...], 0),
                 axis=-1, keepdims=True)
    loc1 = l1_ref[...] + t0
    keep = loc1 < _C
    idx1 = e1 * _C + jnp.minimum(loc1, _C - 1)
    idx1_ref[...] = idx1
    sidx1_ref[...] = jnp.where(keep, idx1, _TRASH)
    w1 = jnp.where(keep, g1_ref[...], 0.0)
    w1_ref[...] = jnp.broadcast_to(w1, (_B2, 16))


def _route1_stage(e1, loc1p, g1, tot0):
    colspec = pl.BlockSpec((_B2, 1), lambda b: (b, 0))
    return pl.pallas_call(
        _route1_body,
        grid=(_NB2,),
        in_specs=[colspec, colspec, colspec,
                  pl.BlockSpec((1, _E), lambda b: (0, 0))],
        out_specs=[colspec, colspec, pl.BlockSpec((_B2, 16), lambda b: (b, 0))],
        out_shape=[jax.ShapeDtypeStruct((_T, 1), jnp.int32),
                   jax.ShapeDtypeStruct((_T, 1), jnp.int32),
                   jax.ShapeDtypeStruct((_T, 16), jnp.float32)],
    )(e1, loc1p, g1, tot0)


_NW = 32
_TPW = _T // _NW
_BCB = 32
_DCB = 16

_SC_MESH = plsc.VectorSubcoreMesh(core_axis_name="c", subcore_axis_name="s")


@functools.partial(
    pl.kernel, mesh=_SC_MESH,
    out_type=jax.ShapeDtypeStruct((_E * _C + 8, _D), jnp.float32),
    scratch_types=[pltpu.VMEM((_BCB, _D), jnp.float32),
                   pltpu.VMEM((_BCB, _D), jnp.float32),
                   pltpu.VMEM((_BCB, _D), jnp.float32),
                   pltpu.VMEM((_TPW // _BCB, _BCB), jnp.int32),
                   pltpu.VMEM((_TPW // _BCB, _BCB), jnp.int32),
                   pltpu.SemaphoreType.DMA,
                   pltpu.SemaphoreType.DMA,
                   pltpu.SemaphoreType.DMA,
                   pltpu.SemaphoreType.DMA,
                   pltpu.SemaphoreType.DMA,
                   pltpu.SemaphoreType.DMA],
)
def _dispatch_sc(x_hbm, sidx0_hbm, sidx1_hbm,
                 disp_hbm,
                 xb0, xb1, xb2, s0full, s1full,
                 seml0, seml1, seml2, sems0, sems1, sems2):
    wid = lax.axis_index("s") * 2 + lax.axis_index("c")
    base = wid * _TPW
    nch = _TPW // _BCB
    xbufs = (xb0, xb1, xb2)
    semls = (seml0, seml1, seml2)
    semss = (sems0, sems1, sems2)
    pltpu.sync_copy(sidx0_hbm.at[wid], s0full)
    pltpu.sync_copy(sidx1_hbm.at[wid], s1full)

    def xload(j):
        return pltpu.async_copy(x_hbm.at[pl.ds(base + j * _BCB, _BCB)],
                                xbufs[j % 3], semls[j % 3])

    xloads = [None] * nch
    scats = [None] * nch
    for j in range(min(2, nch)):
        xloads[j] = xload(j)
    for j in range(nch):
        slot = j % 3
        xloads[j].wait()
        scats[j] = (pltpu.async_copy(xbufs[slot], disp_hbm.at[s0full.at[j]],
                                     semss[slot]),
                    pltpu.async_copy(xbufs[slot], disp_hbm.at[s1full.at[j]],
                                     semss[slot]))
        nj = j + 2
        if nj < nch:
            if nj - 3 >= 0:
                for c in scats[nj - 3]:
                    c.wait()
            xloads[nj] = xload(nj)
    for j in range(max(0, nch - 3), nch):
        for c in scats[j]:
            c.wait()


_NCH = _TPW // _DCB


@functools.partial(
    pl.kernel, mesh=_SC_MESH,
    out_type=jax.ShapeDtypeStruct((_T, _D), jnp.float32),
    scratch_types=[pltpu.VMEM((_DCB, _D), jnp.float32),
                   pltpu.VMEM((_DCB, _D), jnp.float32),
                   pltpu.VMEM((_DCB, _D), jnp.float32),
                   pltpu.VMEM((_DCB, _D), jnp.float32),
                   pltpu.VMEM((_DCB, _D), jnp.float32),
                   pltpu.VMEM((_TPW,), jnp.int32),
                   pltpu.VMEM((_TPW,), jnp.int32),
                   pltpu.VMEM((_TPW * 16,), jnp.float32),
                   pltpu.VMEM((_TPW * 16,), jnp.float32),
                   pltpu.SemaphoreType.DMA,
                   pltpu.SemaphoreType.DMA],
)
def _combine_sc(y_hbm, idx0_hbm, idx1_hbm, w0_hbm, w1_hbm,
                out_hbm,
                r0a, r1a, r0b, r1b, obuf, i0full, i1full, w0full, w1full,
                sema, semb):
    wid = lax.axis_index("s") * 2 + lax.axis_index("c")
    base = wid * _TPW
    pltpu.sync_copy(idx0_hbm.at[pl.ds(base, _TPW)], i0full)
    pltpu.sync_copy(idx1_hbm.at[pl.ds(base, _TPW)], i1full)
    pltpu.sync_copy(w0_hbm.at[pl.ds(base * 16, _TPW * 16)], w0full)
    pltpu.sync_copy(w1_hbm.at[pl.ds(base * 16, _TPW * 16)], w1full)

    r0s = (r0a, r0b)
    r1s = (r1a, r1b)
    sems = (sema, semb)

    def fire(c, slot):
        isl = pl.ds(c * _DCB, _DCB)
        pltpu.async_copy(y_hbm.at[i0full.at[isl]], r0s[slot], sems[slot])
        pltpu.async_copy(y_hbm.at[i1full.at[isl]], r1s[slot], sems[slot])

    def drain(c, slot):
        isl = pl.ds(c * _DCB, _DCB)
        pltpu.make_async_copy(y_hbm.at[i0full.at[isl]], r0s[slot], sems[slot]).wait()
        pltpu.make_async_copy(y_hbm.at[i1full.at[isl]], r1s[slot], sems[slot]).wait()

    fire(0, 0)

    def body2(jj, carry):
        for b in range(2):
            j = jj * 2 + b
            slot = b

            @pl.when(j + 1 < _NCH)
            def _():
                fire(j + 1, 1 - slot)

            drain(j, slot)
            r0 = r0s[slot]
            r1 = r1s[slot]

            def tok(i, cin):
                gi = j * _DCB + i
                w0s = w0full[pl.ds(gi * 16, 16)]
                w1s = w1full[pl.ds(gi * 16, 16)]
                for q in range(_D // 16):
                    sl = pl.ds(q * 16, 16)
                    obuf[i, sl] = w0s * r0[i, sl] + w1s * r1[i, sl]
                return cin

            lax.fori_loop(0, _DCB, tok, 0)
            pltpu.sync_copy(obuf, out_hbm.at[pl.ds(base + j * _DCB, _DCB)])
        return carry

    lax.fori_loop(0, _NCH // 2, body2, 0)


def kernel(x, wg, fc1_w, fc2_w, fc1_b, fc2_b, k):
    ksh = (jnp.asarray(k, jnp.float32) - float(_KTOP)).reshape(1)
    idx0, sidx0, w0, e1, loc1p, g1, tot0, laux = _gate_stage(x, wg, ksh)
    idx1, sidx1, w1 = _route1_stage(e1, loc1p, g1, tot0)

    nchb = _TPW // _BCB
    disp = _dispatch_sc(x, sidx0.reshape(_NW, nchb, _BCB),
                        sidx1.reshape(_NW, nchb, _BCB))
    yflat = _ffn(disp, fc1_w, fc2_w, fc1_b, fc2_b, ksh)
    out = _combine_sc(yflat, idx0.reshape(_T), idx1.reshape(_T),
                      w0.reshape(_T * 16), w1.reshape(_T * 16))
    return out, laux.reshape(())


# speedup vs baseline: 1.0526x; 1.0526x over previous
import functools

import jax
import jax.numpy as jnp
from jax import lax
from jax.experimental import pallas as pl
from jax.experimental.pallas import tpu as pltpu
from jax.experimental.pallas import tpu_sc as plsc

_T, _D, _H, _E = 16384, 768, 768, 8
_KTOP = 2
_C = (_KTOP * _T) // _E
_TRASH = _E * _C
_BG = 1024
_NB = _T // _BG
_BM = 512


def _gate_body(ksh_ref, x_ref, wg_ref,
               idx0_ref, sidx0_ref, w0_ref, e1_ref, loc1p_ref, g1_ref,
               tot0_ref, laux_ref,
               carry0, carry1, me_acc):
    b = pl.program_id(0)

    @pl.when(b == 0)
    def _init():
        carry0[...] = jnp.zeros_like(carry0)
        carry1[...] = jnp.zeros_like(carry1)
        me_acc[...] = jnp.zeros_like(me_acc)

    xb = x_ref[...] + ksh_ref[0]
    logits = jnp.dot(xb, wg_ref[...], preferred_element_type=jnp.float32)
    m = jnp.max(logits, axis=-1, keepdims=True)
    p = jnp.exp(logits - m)
    sc = p / jnp.sum(p, axis=-1, keepdims=True)
    me_acc[...] += jnp.sum(sc, axis=0, keepdims=True)

    lane = lax.broadcasted_iota(jnp.int32, (_BG, _E), 1)
    v0 = jnp.max(sc, axis=-1, keepdims=True)
    i0 = jnp.min(jnp.where(sc == v0, lane, _E), axis=-1, keepdims=True)
    oh0 = (lane == i0).astype(jnp.float32)
    scm = jnp.where(lane == i0, -jnp.inf, sc)
    v1 = jnp.max(scm, axis=-1, keepdims=True)
    i1 = jnp.min(jnp.where(scm == v1, lane, _E), axis=-1, keepdims=True)
    oh1 = (lane == i1).astype(jnp.float32)

    denom = v0 + v1 + 1e-9
    g0 = v0 / denom
    g1 = v1 / denom

    SB = 128
    NS = _BG // SB
    tril_s = (lax.broadcasted_iota(jnp.int32, (SB, SB), 0) >=
              lax.broadcasted_iota(jnp.int32, (SB, SB), 1)).astype(jnp.float32)
    stril = (lax.broadcasted_iota(jnp.int32, (NS, NS), 0) >
             lax.broadcasted_iota(jnp.int32, (NS, NS), 1)).astype(jnp.float32)
    sel = (lax.broadcasted_iota(jnp.int32, (_BG, NS), 0) // SB ==
           lax.broadcasted_iota(jnp.int32, (_BG, NS), 1)).astype(jnp.float32)

    def _cumsum2(oh):
        subcums = []
        subtots = []
        for sblk in range(NS):
            o = oh[sblk * SB:(sblk + 1) * SB, :]
            cs = jnp.dot(tril_s, o, preferred_element_type=jnp.float32)
            subcums.append(cs)
            subtots.append(cs[SB - 1:SB, :])
        cumlocal = jnp.concatenate(subcums, axis=0)
        tots = jnp.concatenate(subtots, axis=0)
        offs = jnp.dot(stril, tots, preferred_element_type=jnp.float32)
        return cumlocal + jnp.dot(sel, offs,
                                  preferred_element_type=jnp.float32)

    cum0 = _cumsum2(oh0)
    cum1 = _cumsum2(oh1)
    loc0 = jnp.sum((cum0 - 1.0 + carry0[...]) * oh0, axis=-1, keepdims=True)
    loc1 = jnp.sum((cum1 - 1.0 + carry1[...]) * oh1, axis=-1, keepdims=True)
    carry0[...] += jnp.sum(oh0, axis=0, keepdims=True)
    carry1[...] += jnp.sum(oh1, axis=0, keepdims=True)

    keep0 = (loc0 < _C).astype(jnp.float32)
    idx0 = i0 * _C + jnp.clip(loc0.astype(jnp.int32), 0, _C - 1)
    idx0_ref[...] = idx0
    sidx0_ref[...] = jnp.where(loc0 < _C, idx0, _TRASH)
    w0_ref[...] = jnp.broadcast_to(g0 * keep0, (_BG, 16))
    e1_ref[...] = i1
    loc1p_ref[...] = loc1.astype(jnp.int32)
    g1_ref[...] = g1

    @pl.when(b == _NB - 1)
    def _fin():
        tot0_ref[...] = carry0[...].astype(jnp.int32)
        laux_ref[...] = jnp.sum((me_acc[...] / _T) * (carry0[...] / _T),
                                axis=-1, keepdims=True) * (_E * _E)


def _gate_stage(x, wg, ksh):
    col_i = jax.ShapeDtypeStruct((_T, 1), jnp.int32)
    col_f = jax.ShapeDtypeStruct((_T, 1), jnp.float32)
    bc_f = jax.ShapeDtypeStruct((_T, 16), jnp.float32)
    colspec = pl.BlockSpec((_BG, 1), lambda b: (b, 0))
    bcspec = pl.BlockSpec((_BG, 16), lambda b: (b, 0))
    fullspec8 = pl.BlockSpec((1, _E), lambda b: (0, 0))
    return pl.pallas_call(
        _gate_body,
        grid=(_NB,),
        in_specs=[
            pl.BlockSpec(memory_space=pltpu.SMEM),
            pl.BlockSpec((_BG, _D), lambda b: (b, 0)),
            pl.BlockSpec((_D, _E), lambda b: (0, 0)),
        ],
        out_specs=[colspec, colspec, bcspec, colspec, colspec, colspec,
                   fullspec8, pl.BlockSpec((1, 1), lambda b: (0, 0))],
        out_shape=[col_i, col_i, bc_f, col_i, col_i, col_f,
                   jax.ShapeDtypeStruct((1, _E), jnp.int32),
                   jax.ShapeDtypeStruct((1, 1), jnp.float32)],
        scratch_shapes=[pltpu.VMEM((1, _E), jnp.float32),
                        pltpu.VMEM((1, _E), jnp.float32),
                        pltpu.VMEM((1, _E), jnp.float32)],
    )(ksh, x, wg)


def _ffn_body(ksh_ref, disp_ref, w1_ref, w2_ref, b1_ref, b2_ref, y_ref,
              wb1_s, wb2_s, bias_s):
    m = pl.program_id(1)

    @pl.when(m == 0)
    def _():
        w1 = w1_ref[0]
        wb1_s[...] = w1.astype(jnp.bfloat16)
        wb2_s[...] = w2_ref[0].astype(jnp.bfloat16)
        bias_s[...] = b1_ref[0] + ksh_ref[0] * jnp.sum(w1, axis=0,
                                                       keepdims=True)

    nh = _BM // 2
    a16a = disp_ref[pl.ds(0, nh), :].astype(jnp.bfloat16)
    a16b = disp_ref[pl.ds(nh, nh), :].astype(jnp.bfloat16)
    ha = jnp.dot(a16a, wb1_s[...], preferred_element_type=jnp.float32)
    hb = jnp.dot(a16b, wb1_s[...], preferred_element_type=jnp.float32)
    h16a = jnp.maximum(ha + bias_s[...], 0.0).astype(jnp.bfloat16)
    h16b = jnp.maximum(hb + bias_s[...], 0.0).astype(jnp.bfloat16)
    ya = jnp.dot(h16a, wb2_s[...], preferred_element_type=jnp.float32)
    yb = jnp.dot(h16b, wb2_s[...], preferred_element_type=jnp.float32)
    y_ref[pl.ds(0, nh), :] = ya + b2_ref[0]
    y_ref[pl.ds(nh, nh), :] = yb + b2_ref[0]


def _ffn(disp, fc1_w, fc2_w, fc1_b, fc2_b, ksh):
    nm = _C // _BM
    return pl.pallas_call(
        _ffn_body,
        grid=(_E, nm),
        in_specs=[
            pl.BlockSpec(memory_space=pltpu.SMEM),
            pl.BlockSpec((_BM, _D), lambda e, m: (e * nm + m, 0)),
            pl.BlockSpec((1, _D, _H), lambda e, m: (e, 0, 0)),
            pl.BlockSpec((1, _H, _D), lambda e, m: (e, 0, 0)),
            pl.BlockSpec((1, 1, _H), lambda e, m: (e, 0, 0)),
            pl.BlockSpec((1, 1, _D), lambda e, m: (e, 0, 0)),
        ],
        out_specs=pl.BlockSpec((_BM, _D), lambda e, m: (e * nm + m, 0)),
        out_shape=jax.ShapeDtypeStruct((_E * _C, _D), jnp.float32),
        scratch_shapes=[pltpu.VMEM((_D, _H), jnp.bfloat16),
                        pltpu.VMEM((_H, _D), jnp.bfloat16),
                        pltpu.VMEM((1, _H), jnp.float32)],
    )(ksh, disp, fc1_w, fc2_w, fc1_b, fc2_b)


_B2 = 2048
_NB2 = _T // _B2


def _route1_body(e1_ref, l1_ref, g1_ref, tot0_ref,
                 idx1_ref, sidx1_ref, w1_ref):
    e1 = e1_ref[...]
    lane = lax.broadcasted_iota(jnp.int32, (_B2, _E), 1)
    t0 = jnp.sum(jnp.where(e1 == lane, tot0_ref[...], 0),
                 axis=-1, keepdims=True)
    loc1 = l1_ref[...] + t0
    keep = loc1 < _C
    idx1 = e1 * _C + jnp.minimum(loc1, _C - 1)
    idx1_ref[...] = idx1
    sidx1_ref[...] = jnp.where(keep, idx1, _TRASH)
    w1 = jnp.where(keep, g1_ref[...], 0.0)
    w1_ref[...] = jnp.broadcast_to(w1, (_B2, 16))


def _route1_stage(e1, loc1p, g1, tot0):
    colspec = pl.BlockSpec((_B2, 1), lambda b: (b, 0))
    return pl.pallas_call(
        _route1_body,
        grid=(_NB2,),
        in_specs=[colspec, colspec, colspec,
                  pl.BlockSpec((1, _E), lambda b: (0, 0))],
        out_specs=[colspec, colspec, pl.BlockSpec((_B2, 16), lambda b: (b, 0))],
        out_shape=[jax.ShapeDtypeStruct((_T, 1), jnp.int32),
                   jax.ShapeDtypeStruct((_T, 1), jnp.int32),
                   jax.ShapeDtypeStruct((_T, 16), jnp.float32)],
    )(e1, loc1p, g1, tot0)


_NW = 32
_TPW = _T // _NW
_BCB = 32
_DCB = 16

_SC_MESH = plsc.VectorSubcoreMesh(core_axis_name="c", subcore_axis_name="s")


@functools.partial(
    pl.kernel, mesh=_SC_MESH,
    out_type=jax.ShapeDtypeStruct((_E * _C + 8, _D), jnp.float32),
    scratch_types=[pltpu.VMEM((_BCB, _D), jnp.float32),
                   pltpu.VMEM((_BCB, _D), jnp.float32),
                   pltpu.VMEM((_BCB, _D), jnp.float32),
                   pltpu.VMEM((_TPW // _BCB, _BCB), jnp.int32),
                   pltpu.VMEM((_TPW // _BCB, _BCB), jnp.int32),
                   pltpu.SemaphoreType.DMA,
                   pltpu.SemaphoreType.DMA,
                   pltpu.SemaphoreType.DMA,
                   pltpu.SemaphoreType.DMA,
                   pltpu.SemaphoreType.DMA,
                   pltpu.SemaphoreType.DMA],
)
def _dispatch_sc(x_hbm, sidx0_hbm, sidx1_hbm,
                 disp_hbm,
                 xb0, xb1, xb2, s0full, s1full,
                 seml0, seml1, seml2, sems0, sems1, sems2):
    wid = lax.axis_index("s") * 2 + lax.axis_index("c")
    base = wid * _TPW
    nch = _TPW // _BCB
    xbufs = (xb0, xb1, xb2)
    semls = (seml0, seml1, seml2)
    semss = (sems0, sems1, sems2)
    pltpu.sync_copy(sidx0_hbm.at[wid], s0full)
    pltpu.sync_copy(sidx1_hbm.at[wid], s1full)

    def xload(j):
        return pltpu.async_copy(x_hbm.at[pl.ds(base + j * _BCB, _BCB)],
                                xbufs[j % 3], semls[j % 3])

    xloads = [None] * nch
    scats = [None] * nch
    for j in range(min(2, nch)):
        xloads[j] = xload(j)
    for j in range(nch):
        slot = j % 3
        xloads[j].wait()
        scats[j] = (pltpu.async_copy(xbufs[slot], disp_hbm.at[s0full.at[j]],
                                     semss[slot]),
                    pltpu.async_copy(xbufs[slot], disp_hbm.at[s1full.at[j]],
                                     semss[slot]))
        nj = j + 2
        if nj < nch:
            if nj - 3 >= 0:
                for c in scats[nj - 3]:
                    c.wait()
            xloads[nj] = xload(nj)
    for j in range(max(0, nch - 3), nch):
        for c in scats[j]:
            c.wait()


_NCH = _TPW // _DCB


@functools.partial(
    pl.kernel, mesh=_SC_MESH,
    out_type=jax.ShapeDtypeStruct((_T, _D), jnp.float32),
    scratch_types=[pltpu.VMEM((_DCB, _D), jnp.float32),
                   pltpu.VMEM((_DCB, _D), jnp.float32),
                   pltpu.VMEM((_DCB, _D), jnp.float32),
                   pltpu.VMEM((_DCB, _D), jnp.float32),
                   pltpu.VMEM((_DCB, _D), jnp.float32),
                   pltpu.VMEM((_TPW,), jnp.int32),
                   pltpu.VMEM((_TPW,), jnp.int32),
                   pltpu.VMEM((_TPW * 16,), jnp.float32),
                   pltpu.VMEM((_TPW * 16,), jnp.float32),
                   pltpu.SemaphoreType.DMA,
                   pltpu.SemaphoreType.DMA],
)
def _combine_sc(y_hbm, idx0_hbm, idx1_hbm, w0_hbm, w1_hbm,
                out_hbm,
                r0a, r1a, r0b, r1b, obuf, i0full, i1full,
                w0full, w1full, sema, semb):
    wid = lax.axis_index("s") * 2 + lax.axis_index("c")
    base = wid * _TPW
    pltpu.sync_copy(idx0_hbm.at[pl.ds(base, _TPW)], i0full)
    pltpu.sync_copy(idx1_hbm.at[pl.ds(base, _TPW)], i1full)
    pltpu.sync_copy(w0_hbm.at[pl.ds(base * 16, _TPW * 16)], w0full)
    pltpu.sync_copy(w1_hbm.at[pl.ds(base * 16, _TPW * 16)], w1full)

    r0s = (r0a, r0b)
    r1s = (r1a, r1b)
    sems = (sema, semb)

    def fire(c, slot):
        isl = pl.ds(c * _DCB, _DCB)
        pltpu.async_copy(y_hbm.at[i0full.at[isl]], r0s[slot], sems[slot])
        pltpu.async_copy(y_hbm.at[i1full.at[isl]], r1s[slot], sems[slot])

    def drain(c, slot):
        isl = pl.ds(c * _DCB, _DCB)
        pltpu.make_async_copy(y_hbm.at[i0full.at[isl]], r0s[slot], sems[slot]).wait()
        pltpu.make_async_copy(y_hbm.at[i1full.at[isl]], r1s[slot], sems[slot]).wait()

    fire(0, 0)

    def body2(jj, carry):
        for b in range(2):
            j = jj * 2 + b
            slot = b

            @pl.when(j + 1 < _NCH)
            def _():
                fire(j + 1, 1 - slot)

            drain(j, slot)
            r0 = r0s[slot]
            r1 = r1s[slot]

            def tok(i, cin):
                gi = j * _DCB + i
                w0s = w0full[pl.ds(gi * 16, 16)]
                w1s = w1full[pl.ds(gi * 16, 16)]
                for q in range(_D // 16):
                    sl = pl.ds(q * 16, 16)
                    obuf[i, sl] = w0s * r0[i, sl] + w1s * r1[i, sl]
                return cin

            lax.fori_loop(0, _DCB, tok, 0)
            pltpu.sync_copy(obuf, out_hbm.at[pl.ds(base + j * _DCB, _DCB)])
        return carry

    lax.fori_loop(0, _NCH // 2, body2, 0)


def kernel(x, wg, fc1_w, fc2_w, fc1_b, fc2_b, k):
    ksh = (jnp.asarray(k, jnp.float32) - float(_KTOP)).reshape(1)
    idx0, sidx0, w0, e1, loc1p, g1, tot0, laux = _gate_stage(x, wg, ksh)
    idx1, sidx1, w1 = _route1_stage(e1, loc1p, g1, tot0)

    nchb = _TPW // _BCB
    disp = _dispatch_sc(x, sidx0.reshape(_NW, nchb, _BCB),
                        sidx1.reshape(_NW, nchb, _BCB))
    yflat = _ffn(disp, fc1_w, fc2_w, fc1_b, fc2_b, ksh)
    out = _combine_sc(yflat, idx0.reshape(_T), idx1.reshape(_T),
                      w0.reshape(_T * 16), w1.reshape(_T * 16))
    return out, laux.reshape(())
